# baseline (speedup 1.0000x reference)
import functools

import jax
import jax.numpy as jnp
from jax import lax
from jax.experimental import pallas as pl
from jax.experimental.pallas import tpu as pltpu
from jax.experimental.pallas import tpu_sc as plsc

E = 64
L = 16
NC = 2
NS = 16
NW = NC * NS
MARGIN = 1.0
ALPHA = 0.2
EPS = 1e-7

SW = 512
E_WIN = 61
U_WIN = 6
E_TW = E_WIN * SW
U_TW = U_WIN * SW
E_N = 1000000
U_N = 100000
E_CAP = 640
U_CAP = 1024
E_MCAP = 32
U_MCAP = 128
SPARE = 16384


def _gather_body(u, i, hp, tp, hn, tn, ut, et, ut_tail, et_tail,
                 rows_u, rows_i, rows_hp, rows_tp, rows_hn, rows_tn,
                 istage,
                 ecol_i, eslot_i, ecol_hp, eslot_hp, ecol_tp, eslot_tp,
                 ecol_hn, eslot_hn, ecol_tn, eslot_tn,
                 ucol, uslot,
                 wbuf0, wbuf1, twbuf_e, twbuf_u,
                 st_i, st_hp, st_tp, st_hn, st_tn, st_u,
                 mcol, mslot, sl_i, sl_hp, sl_tp, sl_hn, sl_tn, slotu,
                 isem, wsem, ssem):
    batch = u.shape[0]
    wid = lax.axis_index("s") * NC + lax.axis_index("c")
    iota = lax.iota(jnp.int32, L)
    zero_v = jnp.zeros((L,), jnp.int32)
    is_last = wid == NW - 1

    e_lo = wid * E_TW
    u_lo = wid * U_TW

    esets = ((i, ecol_i, eslot_i, st_i, rows_i, sl_i),
             (hp, ecol_hp, eslot_hp, st_hp, rows_hp, sl_hp),
             (tp, ecol_tp, eslot_tp, st_tp, rows_tp, sl_tp),
             (hn, ecol_hn, eslot_hn, st_hn, rows_hn, sl_hn),
             (tn, ecol_tn, eslot_tn, st_tn, rows_tn, sl_tn))
    uset = (u, ucol, uslot, st_u, rows_u, slotu)

    CH = 2048
    n_ch = batch // CH
    counts = []
    for arr, pcol, pslot, _, _, _ in esets + (uset,):
        if arr is u:
            lo, span, n_tab = u_lo, U_TW, U_N
        else:
            lo, span, n_tab = e_lo, E_TW, E_N
        lo_v = zero_v + lo
        hi_v = jnp.where(is_last, zero_v + n_tab, lo_v + span)

        def ch_body(ch, off_v, arr=arr, pcol=pcol, pslot=pslot, lo_v=lo_v,
                    hi_v=hi_v):
            c0 = pl.multiple_of(ch * CH, CH)
            pltpu.sync_copy(arr.at[pl.ds(c0, CH)], istage)

            def body(v, off_v):
                vals = istage[pl.ds(v * L, L)]
                mask = (vals >= lo_v) & (vals < hi_v)
                mi = jnp.where(mask, 1, 0).astype(jnp.int32)
                cs = plsc.cumsum(mi)
                pos = off_v + cs - 1
                plsc.store_scatter(pcol, [pos], vals - lo_v, mask=mask)
                slot = (zero_v + ch * CH) + v * L + iota
                plsc.store_scatter(pslot, [pos], slot, mask=mask)
                return off_v + plsc.all_reduce_population_count(mask)

            return lax.fori_loop(0, CH // L, body, off_v)

        off_v = lax.fori_loop(0, n_ch, ch_body, zero_v)
        counts.append(jnp.max(off_v))
    e_counts = counts[:5]
    n_u = counts[5]

    def process_window(wbuf, buf_lo, filt_lo, filt_w, sets, cnts, mcap):
        cps = []
        for (arr, pcol, pslot, stg, rows_out, slotbuf), n_k in zip(sets, cnts):
            lo_v = zero_v + filt_lo
            hi_v = lo_v + filt_w
            blo_v = zero_v + buf_lo
            nvec = (n_k + (L - 1)) // L
            nk_v = zero_v + n_k

            def scan_body(v, off_v, pcol=pcol, pslot=pslot, lo_v=lo_v,
                          hi_v=hi_v, nk_v=nk_v, blo_v=blo_v):
                cols = pcol[pl.ds(v * L, L)]
                slots = pslot[pl.ds(v * L, L)]
                valid = (v * L + iota) < nk_v
                mask = (cols >= lo_v) & (cols < hi_v) & valid
                mi = jnp.where(mask, 1, 0).astype(jnp.int32)
                cs = plsc.cumsum(mi)
                pos = off_v + cs - 1
                plsc.store_scatter(mcol, [pos], cols - blo_v, mask=mask)
                plsc.store_scatter(mslot, [pos], slots, mask=mask)
                return off_v + plsc.all_reduce_population_count(mask)

            off_v = lax.fori_loop(0, nvec, scan_body, zero_v)
            m = jnp.max(off_v)
            m_v = zero_v + m

            def gather_group(g, stg=stg):
                lane = (zero_v + g * L) + iota
                sel = lane < m_v
                colv = jnp.where(sel, mcol[pl.ds(g * L, L)], 0)
                prow = lane

                def d_body(dd, _):
                    for k in range(4):
                        dcol = zero_v + (dd * 4 + k)
                        val = plsc.load_gather(wbuf, [dcol, colv])
                        plsc.store_scatter(stg, [prow, dcol], val)
                    return 0

                lax.fori_loop(0, E // 4, d_body, 0)

            gather_group(0)
            for g in range(1, mcap // L):
                @pl.when(m > g * L)
                def _(g=g):
                    gather_group(g)

            def slot_body(g, _):
                lane = (zero_v + g * L) + iota
                sel = lane < m_v
                slotv = jnp.where(sel, mslot[pl.ds(pl.multiple_of(g * L, L), L)],
                                  zero_v + SPARE)
                slotbuf[pl.ds(pl.multiple_of(g * L, L), L)] = slotv
                return 0

            lax.fori_loop(0, mcap // L, slot_body, 0)

            cps.append(pltpu.async_copy(stg, rows_out.at[slotbuf], ssem))
        return cps

    e_nwin = jnp.where(is_last, E_WIN + 1, E_WIN)

    def e_win_body(w, _):
        wloc = pl.multiple_of(w * SW, SW)
        pltpu.async_copy(et.at[:, pl.ds(e_lo + wloc, SW)], wbuf0, wsem).wait()
        cps = process_window(wbuf0, wloc, wloc, SW, esets, e_counts,
                             E_MCAP)
        for c_ in cps:
            c_.wait()
        return 0

    lax.fori_loop(0, e_nwin, e_win_body, 0)

    u_nwin = jnp.where(is_last, U_WIN + 3, U_WIN)

    def u_win_body(w, _):
        wloc = pl.multiple_of(w * SW, SW)
        pltpu.async_copy(ut.at[:, pl.ds(u_lo + wloc, SW)], wbuf1, wsem).wait()
        cps = process_window(wbuf1, wloc, wloc, SW, (uset,), [n_u],
                             U_MCAP)
        for c_ in cps:
            c_.wait()
        return 0

    lax.fori_loop(0, u_nwin, u_win_body, 0)

    @pl.when(is_last)
    def _():
        ebase = (NW - 1) * E_TW
        pltpu.sync_copy(et_tail, twbuf_e)
        cps = process_window(twbuf_e, 999936 - ebase, 999936 - ebase,
                             E_N - 999936, esets, e_counts, E_MCAP)
        for c_ in cps:
            c_.wait()

        ubase = (NW - 1) * U_TW
        pltpu.async_copy(ut.at[:, pl.ds(99840, 128)],
                         wbuf0.at[:, pl.ds(0, 128)], wsem).wait()
        cps = process_window(wbuf0, 99840 - ubase, 99840 - ubase, 128,
                             (uset,), [n_u], U_MCAP)
        for c_ in cps:
            c_.wait()
        pltpu.sync_copy(ut_tail, twbuf_u)
        cps = process_window(twbuf_u, 99968 - ubase, 99968 - ubase,
                             U_N - 99968, (uset,), [n_u], U_MCAP)
        for c_ in cps:
            c_.wait()


def _make_gather_call(batch):
    mesh = plsc.VectorSubcoreMesh(core_axis_name="c", subcore_axis_name="s")
    f32 = jnp.float32
    i32 = jnp.int32
    rows_t = jax.ShapeDtypeStruct((batch + 128, 2 * E), f32)
    return pl.kernel(
        _gather_body,
        out_type=[rows_t] * 6,
        mesh=mesh,
        compiler_params=pltpu.CompilerParams(
            needs_layout_passes=False, use_tc_tiling_on_sc=True),
        scratch_types=(
            [pltpu.VMEM((2048,), i32)]
            + [pltpu.VMEM((E_CAP,), i32)] * 10
            + [pltpu.VMEM((U_CAP,), i32)] * 2
            + [pltpu.VMEM((E, SW), f32)] * 2
            + [pltpu.VMEM((E, E_N - 999936), f32)]
            + [pltpu.VMEM((E, U_N - 99968), f32)]
            + [pltpu.VMEM((E_MCAP, 2 * E), f32)] * 5
            + [pltpu.VMEM((U_MCAP, 2 * E), f32)]
            + [pltpu.VMEM((U_MCAP,), i32)] * 2
            + [pltpu.VMEM((E_MCAP,), i32)] * 5
            + [pltpu.VMEM((U_MCAP,), i32)]
            + [pltpu.SemaphoreType.DMA] * 3
        ),
    )


def _compute_body(rp, rn, rows_u, rows_i, rows_hp, rows_tp, rows_hn, rows_tn,
                  r_emb,
                  s_out, sqp_out, sqn_out,
                  idx_rp, idx_rn, r_tab,
                  bu0, bi0, bhp0, btp0, bhn0, btn0,
                  bu1, bi1, bhp1, btp1, bhn1, btn1,
                  svec, pvec, nvec, sem0, sem1, isem):
    bpw = svec.shape[0]
    c_rows = bu0.shape[0]
    n_chunks = bpw // c_rows
    groups = c_rows // L

    wid = lax.axis_index("s") * NC + lax.axis_index("c")
    base = wid * bpw
    sl_w = pl.ds(base, bpw)

    stage = [
        pltpu.async_copy(rp.at[sl_w], idx_rp, isem),
        pltpu.async_copy(rn.at[sl_w], idx_rn, isem),
        pltpu.async_copy(r_emb, r_tab, isem),
    ]
    for cp in stage:
        cp.wait()

    bufs = ((bu0, bi0, bhp0, btp0, bhn0, btn0, sem0),
            (bu1, bi1, bhp1, btp1, bhn1, btn1, sem1))

    def issue(c):
        bu, bi, bhp, btp, bhn, btn, sem = bufs[c % 2]
        sl = pl.ds(base + c * c_rows, c_rows)
        return [
            pltpu.async_copy(rows_u.at[sl], bu, sem),
            pltpu.async_copy(rows_i.at[sl], bi, sem),
            pltpu.async_copy(rows_hp.at[sl], bhp, sem),
            pltpu.async_copy(rows_tp.at[sl], btp, sem),
            pltpu.async_copy(rows_hn.at[sl], bhn, sem),
            pltpu.async_copy(rows_tn.at[sl], btn, sem),
        ]

    iota = lax.iota(jnp.int32, L)
    pend = {0: issue(0)}
    for c in range(n_chunks):
        if c + 1 < n_chunks:
            pend[c + 1] = issue(c + 1)
        for cp in pend.pop(c):
            cp.wait()
        bu, bi, bhp, btp, bhn, btn, _ = bufs[c % 2]
        cbase = c * c_rows

        def group_body(g, _, bu=bu, bi=bi, bhp=bhp, btp=btp, bhn=bhn,
                       btn=btn, cbase=cbase):
            goff = pl.multiple_of(g * L, L)
            row = goff + iota
            sl16 = pl.ds(cbase + goff, L)
            rp_v = idx_rp[sl16]
            rn_v = idx_rn[sl16]

            def d_body(dd, accs):
                acc_s, acc_p, acc_n = accs
                for k in range(4):
                    d = dd * 4 + k
                    col = jnp.full((L,), d, jnp.int32)
                    ue = plsc.load_gather(bu, [row, col])
                    ie = plsc.load_gather(bi, [row, col])
                    acc_s = acc_s + ue * ie
                    hpe = plsc.load_gather(bhp, [row, col])
                    tpe = plsc.load_gather(btp, [row, col])
                    rpe = plsc.load_gather(r_tab, [rp_v, col])
                    dp = hpe + rpe - tpe
                    acc_p = acc_p + dp * dp
                    hne = plsc.load_gather(bhn, [row, col])
                    tne = plsc.load_gather(btn, [row, col])
                    rne = plsc.load_gather(r_tab, [rn_v, col])
                    dn = hne + rne - tne
                    acc_n = acc_n + dn * dn
                return acc_s, acc_p, acc_n

            zero = jnp.zeros((L,), jnp.float32)
            acc_s, acc_p, acc_n = lax.fori_loop(0, E // 4, d_body,
                                                (zero, zero, zero))
            svec[sl16] = acc_s
            pvec[sl16] = acc_p
            nvec[sl16] = acc_n
            return 0

        lax.fori_loop(0, groups, group_body, 0)

    pltpu.sync_copy(svec, s_out.at[sl_w])
    pltpu.sync_copy(pvec, sqp_out.at[sl_w])
    pltpu.sync_copy(nvec, sqn_out.at[sl_w])


def _make_compute_call(batch):
    bpw = batch // NW
    c_rows = 64
    mesh = plsc.VectorSubcoreMesh(core_axis_name="c", subcore_axis_name="s")
    f32 = jnp.float32
    return pl.kernel(
        _compute_body,
        out_type=[jax.ShapeDtypeStruct((batch,), f32)] * 3,
        mesh=mesh,
        compiler_params=pltpu.CompilerParams(
            needs_layout_passes=False, use_tc_tiling_on_sc=True),
        scratch_types=(
            [pltpu.VMEM((bpw,), jnp.int32)] * 2
            + [pltpu.VMEM((64, E), f32)]
            + [pltpu.VMEM((c_rows, 2 * E), f32)] * 12
            + [pltpu.VMEM((bpw,), f32)] * 3
            + [pltpu.SemaphoreType.DMA] * 3
        ),
    )


def _finish_body(y_ref, s_ref, p_ref, n_ref, o_ref):
    s = s_ref[...]
    yp = jnp.clip(1.0 / (1.0 + jnp.exp(-s)), EPS, 1.0 - EPS)
    yv = y_ref[...]
    bce = -(yv * jnp.log(yp) + (1.0 - yv) * jnp.log(1.0 - yp))
    ypos = jnp.sqrt(p_ref[...])
    yneg = jnp.sqrt(n_ref[...])
    hinge = jnp.maximum(ypos - yneg + MARGIN, 0.0)
    n = s.shape[0] * s.shape[1]
    o_ref[0, 0] = jnp.sum(bce) / n + ALPHA * jnp.sum(hinge)


def kernel(u, i, y, h_pos, r_pos, t_pos, h_neg, r_neg, t_neg, u_emb, e_emb, r_emb):
    batch = u.shape[0]
    et = e_emb.T
    ut = u_emb.T

    ut_tail = u_emb[99968:, :].T
    et_tail = e_emb[999936:, :].T
    gather_call = _make_gather_call(batch)
    rows = gather_call(u.astype(jnp.int32), i.astype(jnp.int32),
                       h_pos.astype(jnp.int32), t_pos.astype(jnp.int32),
                       h_neg.astype(jnp.int32), t_neg.astype(jnp.int32),
                       ut, et, ut_tail, et_tail)
    rows_u, rows_i, rows_hp, rows_tp, rows_hn, rows_tn = rows

    compute_call = _make_compute_call(batch)
    s, sqp, sqn = compute_call(r_pos.astype(jnp.int32), r_neg.astype(jnp.int32),
                               rows_u, rows_i, rows_hp, rows_tp, rows_hn,
                               rows_tn, r_emb)
    nrow = batch // 128
    shape2d = (nrow, 128)
    out = pl.pallas_call(
        _finish_body,
        out_shape=jax.ShapeDtypeStruct((1, 1), jnp.float32),
        out_specs=pl.BlockSpec(memory_space=pltpu.SMEM),
    )(y.reshape(shape2d), s.reshape(shape2d), sqp.reshape(shape2d),
      sqn.reshape(shape2d))
    return out[0, 0]
